# baseline (speedup 1.0000x reference)
import jax
import jax.numpy as jnp
from jax.experimental import pallas as pl
from jax.experimental.pallas import tpu as pltpu

_C = 0.7213475204444817


def _layer_body(h_ref, w_ref, vs_ref, vrt_ref, a_ref, o_ref, mask_out_ref,
                mx_scr, fs_scr, fd_scr, cs_scr):
    i = pl.program_id(0)
    n = a_ref.shape[1]
    br = a_ref.shape[0]
    d = w_ref.shape[1]

    @pl.when(i == 0)
    def _prep():
        m = jnp.dot(h_ref[...], w_ref[...], preferred_element_type=jnp.float32)
        mx_scr[:, :d] = m.astype(jnp.bfloat16)
        mx_scr[:, d:] = jnp.ones_like(mx_scr[:, d:])
        fs = jnp.dot(m, vs_ref[...], preferred_element_type=jnp.float32) * 0.5
        fd = jax.lax.dot_general(vrt_ref[...], m, (((1,), (1,)), ((), ())),
                                 preferred_element_type=jnp.float32) * 0.5
        fs_scr[...] = fs.astype(jnp.bfloat16)
        fd_scr[...] = fd.astype(jnp.bfloat16)
        cs_scr[...] = jnp.sum(m, axis=0, keepdims=True)

    @pl.when(i > 0)
    def _attn():
        r0 = (i - 1) * br
        if mask_out_ref is not None:
            a = a_ref[...]
            mask_out_ref[...] = a.astype(jnp.int8)
            mk = a.astype(jnp.float32).astype(jnp.bfloat16)
        else:
            mk = jax.lax.bitcast_convert_type(
                a_ref[...].astype(jnp.int16) * jnp.int16(16256), jnp.bfloat16)
        fs = fs_scr[pl.ds(r0, br), :]
        ck = min(2048, n)
        nd = None
        for k in range(n // ck):
            xh = fs + fd_scr[:, k * ck:(k + 1) * ck]
            th = jnp.tanh(xh)
            e = jnp.exp2(th * jnp.bfloat16(_C)) * mk[:, k * ck:(k + 1) * ck]
            pd = jnp.dot(e, mx_scr[k * ck:(k + 1) * ck, :],
                         preferred_element_type=jnp.float32)
            nd = pd if nd is None else nd + pd
        num = nd[:, :d]
        denom = nd[:, d:]
        mean = cs_scr[...] * (1.0 / n)
        out = jnp.where(denom > 0.0, num / denom, mean)
        s = jnp.sum(out * out, axis=1, keepdims=True)
        o_ref[...] = out * jax.lax.rsqrt(s + 1e-30)


def _layer_body_emit(h_ref, w_ref, vs_ref, vrt_ref, a_ref, o_ref, mask_ref,
                     mx_scr, fs_scr, fd_scr, cs_scr):
    _layer_body(h_ref, w_ref, vs_ref, vrt_ref, a_ref, o_ref, mask_ref,
                mx_scr, fs_scr, fd_scr, cs_scr)


def _layer_body_noemit(h_ref, w_ref, vs_ref, vrt_ref, a_ref, o_ref,
                       mx_scr, fs_scr, fd_scr, cs_scr):
    _layer_body(h_ref, w_ref, vs_ref, vrt_ref, a_ref, o_ref, None,
                mx_scr, fs_scr, fd_scr, cs_scr)


def _gat_layer(H, adj, W, vs, vr, emit_mask, block_rows):
    n, d_in = H.shape
    d_out = W.shape[1]
    br = min(block_rows, n)
    vrt = vr.reshape(1, d_out)
    grid = (1 + n // br,)
    zero = lambda i: (0, 0)
    rowblk = lambda i: (jnp.maximum(i - 1, 0), 0)
    in_specs = [
        pl.BlockSpec((n, d_in), zero),
        pl.BlockSpec((d_in, d_out), zero),
        pl.BlockSpec((d_out, 1), zero),
        pl.BlockSpec((1, d_out), zero),
        pl.BlockSpec((br, n), rowblk),
    ]
    scratch_shapes = [
        pltpu.VMEM((n, d_out + 1), jnp.bfloat16),
        pltpu.VMEM((n, 1), jnp.bfloat16),
        pltpu.VMEM((1, n), jnp.bfloat16),
        pltpu.VMEM((1, d_out), jnp.float32),
    ]
    out_spec = pl.BlockSpec((br, d_out), rowblk)
    if emit_mask:
        out, mask8 = pl.pallas_call(
            _layer_body_emit,
            grid=grid,
            in_specs=in_specs,
            out_specs=[out_spec, pl.BlockSpec((br, n), rowblk)],
            out_shape=[
                jax.ShapeDtypeStruct((n, d_out), jnp.float32),
                jax.ShapeDtypeStruct((n, n), jnp.int8),
            ],
            scratch_shapes=scratch_shapes,
        )(H, W, vs, vrt, adj)
        return out, mask8
    out = pl.pallas_call(
        _layer_body_noemit,
        grid=grid,
        in_specs=in_specs,
        out_specs=out_spec,
        out_shape=jax.ShapeDtypeStruct((n, d_out), jnp.float32),
        scratch_shapes=scratch_shapes,
    )(H, W, vs, vrt, adj)
    return out, None


def kernel(X, A, W0, vs0, vr0, W1, vs1, vr1, W2, vs2, vr2):
    H, mask8 = _gat_layer(X, A, W0, vs0, vr0, True, 512)
    H, _ = _gat_layer(H, mask8, W1, vs1, vr1, False, 2048)
    H, _ = _gat_layer(H, mask8, W2, vs2, vr2, False, 2048)
    return H


# speedup vs baseline: 1.1195x; 1.1195x over previous
import jax
import jax.numpy as jnp
from jax.experimental import pallas as pl
from jax.experimental.pallas import tpu as pltpu

_C = 0.7213475204444817


def _layer_body(h_ref, w_ref, vs_ref, vrt_ref, a_ref, o_ref, mask_out_ref,
                mx_scr, fs_scr, fd_scr, cs_scr, nd_scr):
    i = pl.program_id(0)
    nb = pl.num_programs(0) - 2
    n = a_ref.shape[1]
    br = a_ref.shape[0]
    d = w_ref.shape[1]

    @pl.when(i == 0)
    def _prep():
        m = jnp.dot(h_ref[...], w_ref[...], preferred_element_type=jnp.float32)
        mx_scr[:, :d] = m.astype(jnp.bfloat16)
        mx_scr[:, d:] = jnp.ones_like(mx_scr[:, d:])
        fs = jnp.dot(m, vs_ref[...], preferred_element_type=jnp.float32) * 0.5
        fd = jax.lax.dot_general(vrt_ref[...], m, (((1,), (1,)), ((), ())),
                                 preferred_element_type=jnp.float32) * 0.5
        fs_scr[...] = fs.astype(jnp.bfloat16)
        fd_scr[...] = fd.astype(jnp.bfloat16)
        cs_scr[...] = jnp.sum(m, axis=0, keepdims=True)

    @pl.when(i >= 2)
    def _epi():
        nd = nd_scr[...]
        num = nd[:, :d]
        denom = nd[:, d:]
        mean = cs_scr[...] * (1.0 / n)
        out = jnp.where(denom > 0.0, num / denom, mean)
        s = jnp.sum(out * out, axis=1, keepdims=True)
        o_ref[...] = out * jax.lax.rsqrt(s + 1e-30)

    @pl.when(jnp.logical_and(i >= 1, i <= nb))
    def _attn():
        r0 = (i - 1) * br
        if mask_out_ref is not None:
            a = a_ref[...]
            mask_out_ref[...] = a.astype(jnp.int8)
            mk = a.astype(jnp.float32).astype(jnp.bfloat16)
        else:
            mk = jax.lax.bitcast_convert_type(
                a_ref[...].astype(jnp.int16) * jnp.int16(16256), jnp.bfloat16)
        fs = fs_scr[pl.ds(r0, br), :]
        ck = min(2048, n)
        nd = None
        for k in range(n // ck):
            xh = fs + fd_scr[:, k * ck:(k + 1) * ck]
            th = jnp.tanh(xh)
            e = jnp.exp2(th * jnp.bfloat16(_C)) * mk[:, k * ck:(k + 1) * ck]
            pd = jnp.dot(e, mx_scr[k * ck:(k + 1) * ck, :],
                         preferred_element_type=jnp.float32)
            nd = pd if nd is None else nd + pd
        nd_scr[...] = nd


def _layer_body_emit(h_ref, w_ref, vs_ref, vrt_ref, a_ref, o_ref, mask_ref,
                     mx_scr, fs_scr, fd_scr, cs_scr, nd_scr):
    _layer_body(h_ref, w_ref, vs_ref, vrt_ref, a_ref, o_ref, mask_ref,
                mx_scr, fs_scr, fd_scr, cs_scr, nd_scr)


def _layer_body_noemit(h_ref, w_ref, vs_ref, vrt_ref, a_ref, o_ref,
                       mx_scr, fs_scr, fd_scr, cs_scr, nd_scr):
    _layer_body(h_ref, w_ref, vs_ref, vrt_ref, a_ref, o_ref, None,
                mx_scr, fs_scr, fd_scr, cs_scr, nd_scr)


def _gat_layer(H, adj, W, vs, vr, emit_mask, block_rows):
    n, d_in = H.shape
    d_out = W.shape[1]
    br = min(block_rows, n)
    vrt = vr.reshape(1, d_out)
    nb = n // br
    grid = (nb + 2,)
    zero = lambda i: (0, 0)
    rowblk = lambda i: (jnp.clip(i - 1, 0, nb - 1), 0)
    outblk = lambda i: (jnp.clip(i - 2, 0, nb - 1), 0)
    in_specs = [
        pl.BlockSpec((n, d_in), zero),
        pl.BlockSpec((d_in, d_out), zero),
        pl.BlockSpec((d_out, 1), zero),
        pl.BlockSpec((1, d_out), zero),
        pl.BlockSpec((br, n), rowblk),
    ]
    scratch_shapes = [
        pltpu.VMEM((n, d_out + 1), jnp.bfloat16),
        pltpu.VMEM((n, 1), jnp.bfloat16),
        pltpu.VMEM((1, n), jnp.bfloat16),
        pltpu.VMEM((1, d_out), jnp.float32),
        pltpu.VMEM((br, d_out + 1), jnp.float32),
    ]
    out_spec = pl.BlockSpec((br, d_out), outblk)
    if emit_mask:
        out, mask8 = pl.pallas_call(
            _layer_body_emit,
            grid=grid,
            in_specs=in_specs,
            out_specs=[out_spec, pl.BlockSpec((br, n), rowblk)],
            out_shape=[
                jax.ShapeDtypeStruct((n, d_out), jnp.float32),
                jax.ShapeDtypeStruct((n, n), jnp.int8),
            ],
            scratch_shapes=scratch_shapes,
        )(H, W, vs, vrt, adj)
        return out, mask8
    out = pl.pallas_call(
        _layer_body_noemit,
        grid=grid,
        in_specs=in_specs,
        out_specs=out_spec,
        out_shape=jax.ShapeDtypeStruct((n, d_out), jnp.float32),
        scratch_shapes=scratch_shapes,
    )(H, W, vs, vrt, adj)
    return out, None


def kernel(X, A, W0, vs0, vr0, W1, vs1, vr1, W2, vs2, vr2):
    H, mask8 = _gat_layer(X, A, W0, vs0, vr0, True, 512)
    H, _ = _gat_layer(H, mask8, W1, vs1, vr1, False, 1024)
    H, _ = _gat_layer(H, mask8, W2, vs2, vr2, False, 1024)
    return H


# speedup vs baseline: 1.1230x; 1.0032x over previous
import jax
import jax.numpy as jnp
from jax.experimental import pallas as pl
from jax.experimental.pallas import tpu as pltpu

_C = 0.7213475204444817


def _layer_body(h_ref, w_ref, vs_ref, vrt_ref, a_ref, o_ref, mask_out_ref,
                mx_scr, fs_scr, fd_scr, cs_scr, nd_scr):
    i = pl.program_id(0)
    nb = pl.num_programs(0) - 2
    n = a_ref.shape[1]
    br = a_ref.shape[0]
    d = w_ref.shape[1]

    @pl.when(i == 0)
    def _prep():
        m = jnp.dot(h_ref[...], w_ref[...], preferred_element_type=jnp.float32)
        mx_scr[:, :d] = m.astype(jnp.bfloat16)
        mx_scr[:, d:] = jnp.ones_like(mx_scr[:, d:])
        fs = jnp.dot(m, vs_ref[...], preferred_element_type=jnp.float32) * 0.5
        fd = jax.lax.dot_general(vrt_ref[...], m, (((1,), (1,)), ((), ())),
                                 preferred_element_type=jnp.float32) * 0.5
        fs_scr[...] = fs.astype(jnp.bfloat16)
        fd_scr[...] = fd.astype(jnp.bfloat16)
        cs_scr[...] = jnp.sum(m, axis=0, keepdims=True)

    @pl.when(i >= 2)
    def _epi():
        nd = nd_scr[...]
        num = nd[:, :d]
        denom = nd[:, d:]
        mean = cs_scr[...] * (1.0 / n)
        out = jnp.where(denom > 0.0, num / denom, mean)
        s = jnp.sum(out * out, axis=1, keepdims=True)
        o_ref[...] = out * jax.lax.rsqrt(s + 1e-30)

    @pl.when(jnp.logical_and(i >= 1, i <= nb))
    def _attn():
        r0 = (i - 1) * br
        if mask_out_ref is not None:
            a = a_ref[...]
            mask_out_ref[...] = a.astype(jnp.int8)
            mk = a.astype(jnp.float32).astype(jnp.bfloat16)
        else:
            mk = jax.lax.bitcast_convert_type(
                a_ref[...].astype(jnp.int16) * jnp.int16(16256), jnp.bfloat16)
        fs = fs_scr[pl.ds(r0, br), :]
        ck = min(1024, n)
        nd = None
        for k in range(n // ck):
            xh = fs + fd_scr[:, k * ck:(k + 1) * ck]
            th = jnp.tanh(xh)
            e = jnp.exp2(th * jnp.bfloat16(_C)) * mk[:, k * ck:(k + 1) * ck]
            pd = jnp.dot(e, mx_scr[k * ck:(k + 1) * ck, :],
                         preferred_element_type=jnp.float32)
            nd = pd if nd is None else nd + pd
        nd_scr[...] = nd


def _layer_body_emit(h_ref, w_ref, vs_ref, vrt_ref, a_ref, o_ref, mask_ref,
                     mx_scr, fs_scr, fd_scr, cs_scr, nd_scr):
    _layer_body(h_ref, w_ref, vs_ref, vrt_ref, a_ref, o_ref, mask_ref,
                mx_scr, fs_scr, fd_scr, cs_scr, nd_scr)


def _layer_body_noemit(h_ref, w_ref, vs_ref, vrt_ref, a_ref, o_ref,
                       mx_scr, fs_scr, fd_scr, cs_scr, nd_scr):
    _layer_body(h_ref, w_ref, vs_ref, vrt_ref, a_ref, o_ref, None,
                mx_scr, fs_scr, fd_scr, cs_scr, nd_scr)


def _gat_layer(H, adj, W, vs, vr, emit_mask, block_rows):
    n, d_in = H.shape
    d_out = W.shape[1]
    br = min(block_rows, n)
    vrt = vr.reshape(1, d_out)
    nb = n // br
    grid = (nb + 2,)
    zero = lambda i: (0, 0)
    rowblk = lambda i: (jnp.clip(i - 1, 0, nb - 1), 0)
    outblk = lambda i: (jnp.clip(i - 2, 0, nb - 1), 0)
    in_specs = [
        pl.BlockSpec((n, d_in), zero),
        pl.BlockSpec((d_in, d_out), zero),
        pl.BlockSpec((d_out, 1), zero),
        pl.BlockSpec((1, d_out), zero),
        pl.BlockSpec((br, n), rowblk),
    ]
    scratch_shapes = [
        pltpu.VMEM((n, d_out + 1), jnp.bfloat16),
        pltpu.VMEM((n, 1), jnp.bfloat16),
        pltpu.VMEM((1, n), jnp.bfloat16),
        pltpu.VMEM((1, d_out), jnp.float32),
        pltpu.VMEM((br, d_out + 1), jnp.float32),
    ]
    out_spec = pl.BlockSpec((br, d_out), outblk)
    if emit_mask:
        out, mask8 = pl.pallas_call(
            _layer_body_emit,
            grid=grid,
            in_specs=in_specs,
            out_specs=[out_spec, pl.BlockSpec((br, n), rowblk)],
            out_shape=[
                jax.ShapeDtypeStruct((n, d_out), jnp.float32),
                jax.ShapeDtypeStruct((n, n), jnp.int8),
            ],
            scratch_shapes=scratch_shapes,
        )(H, W, vs, vrt, adj)
        return out, mask8
    out = pl.pallas_call(
        _layer_body_noemit,
        grid=grid,
        in_specs=in_specs,
        out_specs=out_spec,
        out_shape=jax.ShapeDtypeStruct((n, d_out), jnp.float32),
        scratch_shapes=scratch_shapes,
    )(H, W, vs, vrt, adj)
    return out, None


def kernel(X, A, W0, vs0, vr0, W1, vs1, vr1, W2, vs2, vr2):
    H, mask8 = _gat_layer(X, A, W0, vs0, vr0, True, 512)
    H, _ = _gat_layer(H, mask8, W1, vs1, vr1, False, 1024)
    H, _ = _gat_layer(H, mask8, W2, vs2, vr2, False, 1024)
    return H
